# baseline (device time: 26776 ns/iter reference)
import jax
import jax.numpy as jnp
from jax import lax
from jax.experimental import pallas as pl
from jax.experimental.pallas import tpu as pltpu

N_DEV = 16


def kernel(x, Wq, K_ext, V_ext, Wo):
    B, Sq, Din = x.shape
    _, Skv, Hl, Dh = K_ext.shape
    Dout = Wo.shape[1]
    F = Hl * Dh
    R = B * Sq
    SEG = R // N_DEV
    CPB = Sq // SEG

    def body(x_ref, wq_ref, k_ref, v_ref, wo_ref, out_ref,
             x_vmem, wq_vmem, wo_vmem, kv_vmem, acc_ref, recv_ref, recv2_ref,
             in_sems, kv_sems, out_sem,
             p1_send_sems, p1_recv_sems, p2_send_sems, p2_recv_sems):
        my = lax.axis_index("i")

        x_dma = pltpu.make_async_copy(x_ref, x_vmem, in_sems.at[0])
        wq_dma = pltpu.make_async_copy(
            wq_ref.at[:, pl.ds(my * F, F)], wq_vmem, in_sems.at[1])
        wo_dma = pltpu.make_async_copy(
            wo_ref.at[pl.ds(my * F, F), :], wo_vmem, in_sems.at[2])
        x_dma.start()
        wq_dma.start()
        wo_dma.start()
        kv_dmas = []
        for i, ref in enumerate((k_ref, v_ref)):
            for b in range(B):
                for h in range(Hl):
                    dma = pltpu.make_async_copy(
                        ref.at[b, :, h, :], kv_vmem.at[i, b, h],
                        kv_sems.at[i, b, h])
                    dma.start()
                    kv_dmas.append(dma)

        barrier = pltpu.get_barrier_semaphore()
        for j in range(1, N_DEV):
            pl.semaphore_signal(
                barrier, inc=1,
                device_id=(lax.rem(my + j, N_DEV),),
                device_id_type=pl.DeviceIdType.MESH,
            )
        pl.semaphore_wait(barrier, N_DEV - 1)

        x_dma.wait()
        wq_dma.wait()
        for dma in kv_dmas:
            dma.wait()
        ctx_rows = []
        for b in range(B):
            qb = jnp.dot(x_vmem[b], wq_vmem[...],
                         preferred_element_type=jnp.float32)
            cols = []
            for h in range(Hl):
                s = lax.dot_general(
                    qb[:, h * Dh:(h + 1) * Dh], kv_vmem[0, b, h],
                    (((1,), (1,)), ((), ())),
                    preferred_element_type=jnp.float32)
                e = jnp.exp(s * 0.125)
                w = e / jnp.sum(e, axis=-1, keepdims=True)
                cols.append(jnp.dot(w, kv_vmem[1, b, h],
                                    preferred_element_type=jnp.float32))
            ctx_rows.append(jnp.concatenate(cols, axis=-1))
        ctx = jnp.concatenate(ctx_rows, axis=0)
        wo_dma.wait()
        partial = jnp.dot(ctx, wo_vmem[...],
                          preferred_element_type=jnp.float32)
        for b in range(B):
            acc_ref[b] = partial[b * Sq:(b + 1) * Sq, :]

        p1_sends = []
        for j in range(1, N_DEV):
            t = lax.rem(my + j, N_DEV)
            rdma = pltpu.make_async_remote_copy(
                src_ref=acc_ref.at[t // CPB, pl.ds(lax.rem(t, CPB) * SEG, SEG), :],
                dst_ref=recv_ref.at[my],
                send_sem=p1_send_sems.at[t],
                recv_sem=p1_recv_sems.at[my],
                device_id=(t,),
                device_id_type=pl.DeviceIdType.MESH,
            )
            rdma.start()
            p1_sends.append(rdma)
        recv_ref[my] = acc_ref[my // CPB, pl.ds(lax.rem(my, CPB) * SEG, SEG), :]

        for j in range(1, N_DEV):
            src = lax.rem(my + j, N_DEV)
            pltpu.make_async_remote_copy(
                src_ref=recv_ref.at[src],
                dst_ref=recv_ref.at[src],
                send_sem=p1_send_sems.at[src],
                recv_sem=p1_recv_sems.at[src],
                device_id=(src,),
                device_id_type=pl.DeviceIdType.MESH,
            ).wait_recv()
        for rdma in p1_sends:
            rdma.wait_send()

        reduced = jnp.sum(recv_ref[...], axis=0)
        acc_ref[my // CPB, pl.ds(lax.rem(my, CPB) * SEG, SEG), :] = reduced
        recv2_ref[my] = reduced

        p2_sends = []
        for j in range(1, N_DEV):
            t = lax.rem(my + j, N_DEV)
            rdma = pltpu.make_async_remote_copy(
                src_ref=recv2_ref.at[my],
                dst_ref=recv2_ref.at[my],
                send_sem=p2_send_sems.at[t],
                recv_sem=p2_recv_sems.at[my],
                device_id=(t,),
                device_id_type=pl.DeviceIdType.MESH,
            )
            rdma.start()
            p2_sends.append(rdma)
        for j in range(1, N_DEV):
            src = lax.rem(my + j, N_DEV)
            pltpu.make_async_remote_copy(
                src_ref=recv2_ref.at[src],
                dst_ref=recv2_ref.at[src],
                send_sem=p2_send_sems.at[src],
                recv_sem=p2_recv_sems.at[src],
                device_id=(src,),
                device_id_type=pl.DeviceIdType.MESH,
            ).wait_recv()
            acc_ref[src // CPB, pl.ds(lax.rem(src, CPB) * SEG, SEG), :] = (
                recv2_ref[src])
        for rdma in p2_sends:
            rdma.wait_send()

        out_dma = pltpu.make_async_copy(acc_ref, out_ref, out_sem)
        out_dma.start()
        out_dma.wait()

    hbm = pltpu.MemorySpace.HBM
    x_c = pltpu.with_memory_space_constraint(x, hbm)
    wq_c = pltpu.with_memory_space_constraint(Wq, hbm)
    wo_c = pltpu.with_memory_space_constraint(Wo, hbm)
    return pl.pallas_call(
        body,
        out_shape=jax.ShapeDtypeStruct((B, Sq, Dout), jnp.float32),
        in_specs=[
            pl.BlockSpec(memory_space=pl.ANY),
            pl.BlockSpec(memory_space=pl.ANY),
            pl.BlockSpec(memory_space=pl.ANY),
            pl.BlockSpec(memory_space=pl.ANY),
            pl.BlockSpec(memory_space=pl.ANY),
        ],
        out_specs=pl.BlockSpec(memory_space=pl.ANY),
        scratch_shapes=[
            pltpu.VMEM((B, Sq, Din), jnp.float32),
            pltpu.VMEM((Din, F), jnp.float32),
            pltpu.VMEM((F, Dout), jnp.float32),
            pltpu.VMEM((2, B, Hl, Skv, Dh), jnp.float32),
            pltpu.VMEM((B, Sq, Dout), jnp.float32),
            pltpu.VMEM((N_DEV, SEG, Dout), jnp.float32),
            pltpu.VMEM((N_DEV, SEG, Dout), jnp.float32),
            pltpu.SemaphoreType.DMA((3,)),
            pltpu.SemaphoreType.DMA((2, B, Hl)),
            pltpu.SemaphoreType.DMA,
            pltpu.SemaphoreType.DMA((N_DEV,)),
            pltpu.SemaphoreType.DMA((N_DEV,)),
            pltpu.SemaphoreType.DMA((N_DEV,)),
            pltpu.SemaphoreType.DMA((N_DEV,)),
        ],
        compiler_params=pltpu.CompilerParams(collective_id=0),
    )(x_c, wq_c,
      pltpu.with_memory_space_constraint(K_ext, hbm),
      pltpu.with_memory_space_constraint(V_ext, hbm), wo_c)


# device time: 24912 ns/iter; 1.0748x vs baseline; 1.0748x over previous
import jax
import jax.numpy as jnp
from jax import lax
from jax.experimental import pallas as pl
from jax.experimental.pallas import tpu as pltpu

N_DEV = 16


def kernel(x, Wq, K_ext, V_ext, Wo):
    B, Sq, Din = x.shape
    _, Skv, Hl, Dh = K_ext.shape
    Dout = Wo.shape[1]
    F = Hl * Dh
    R = B * Sq
    SEG = R // N_DEV
    CPB = Sq // SEG

    def body(x_ref, wq_ref, k_ref, v_ref, wo_ref, out_ref,
             x_vmem, wq_vmem, wo_vmem, acc_ref, recv_ref, recv2_ref,
             in_sems, out_sem,
             p1_send_sems, p1_recv_sems, p2_send_sems, p2_recv_sems):
        my = lax.axis_index("i")

        x_dma = pltpu.make_async_copy(x_ref, x_vmem, in_sems.at[0])
        wq_dma = pltpu.make_async_copy(
            wq_ref.at[:, pl.ds(my * F, F)], wq_vmem, in_sems.at[1])
        wo_dma = pltpu.make_async_copy(
            wo_ref.at[pl.ds(my * F, F), :], wo_vmem, in_sems.at[2])
        x_dma.start()
        wq_dma.start()
        wo_dma.start()

        barrier = pltpu.get_barrier_semaphore()
        for j in range(1, N_DEV):
            pl.semaphore_signal(
                barrier, inc=1,
                device_id=(lax.rem(my + j, N_DEV),),
                device_id_type=pl.DeviceIdType.MESH,
            )
        pl.semaphore_wait(barrier, N_DEV - 1)

        x_dma.wait()
        wq_dma.wait()
        ctx_rows = []
        for b in range(B):
            qb = jnp.dot(x_vmem[b], wq_vmem[...],
                         preferred_element_type=jnp.float32)
            kb = k_ref[b]
            vb = v_ref[b]
            cols = []
            for h in range(Hl):
                s = lax.dot_general(
                    qb[:, h * Dh:(h + 1) * Dh], kb[:, h * Dh:(h + 1) * Dh],
                    (((1,), (1,)), ((), ())),
                    preferred_element_type=jnp.float32)
                e = jnp.exp(s * 0.125)
                w = e / jnp.sum(e, axis=-1, keepdims=True)
                cols.append(jnp.dot(w, vb[:, h * Dh:(h + 1) * Dh],
                                    preferred_element_type=jnp.float32))
            ctx_rows.append(jnp.concatenate(cols, axis=-1))
        ctx = jnp.concatenate(ctx_rows, axis=0)
        wo_dma.wait()
        partial = jnp.dot(ctx, wo_vmem[...],
                          preferred_element_type=jnp.float32)
        for b in range(B):
            acc_ref[b] = partial[b * Sq:(b + 1) * Sq, :]

        p1_sends = []
        for j in range(1, N_DEV):
            t = lax.rem(my + j, N_DEV)
            rdma = pltpu.make_async_remote_copy(
                src_ref=acc_ref.at[t // CPB, pl.ds(lax.rem(t, CPB) * SEG, SEG), :],
                dst_ref=recv_ref.at[my],
                send_sem=p1_send_sems.at[t],
                recv_sem=p1_recv_sems.at[my],
                device_id=(t,),
                device_id_type=pl.DeviceIdType.MESH,
            )
            rdma.start()
            p1_sends.append(rdma)
        recv_ref[my] = acc_ref[my // CPB, pl.ds(lax.rem(my, CPB) * SEG, SEG), :]

        for j in range(1, N_DEV):
            src = lax.rem(my + j, N_DEV)
            pltpu.make_async_remote_copy(
                src_ref=recv_ref.at[src],
                dst_ref=recv_ref.at[src],
                send_sem=p1_send_sems.at[src],
                recv_sem=p1_recv_sems.at[src],
                device_id=(src,),
                device_id_type=pl.DeviceIdType.MESH,
            ).wait_recv()
        for rdma in p1_sends:
            rdma.wait_send()

        reduced = jnp.sum(recv_ref[...], axis=0)
        acc_ref[my // CPB, pl.ds(lax.rem(my, CPB) * SEG, SEG), :] = reduced
        recv2_ref[my] = reduced

        p2_sends = []
        for j in range(1, N_DEV):
            t = lax.rem(my + j, N_DEV)
            rdma = pltpu.make_async_remote_copy(
                src_ref=recv2_ref.at[my],
                dst_ref=recv2_ref.at[my],
                send_sem=p2_send_sems.at[t],
                recv_sem=p2_recv_sems.at[my],
                device_id=(t,),
                device_id_type=pl.DeviceIdType.MESH,
            )
            rdma.start()
            p2_sends.append(rdma)
        for j in range(1, N_DEV):
            src = lax.rem(my + j, N_DEV)
            pltpu.make_async_remote_copy(
                src_ref=recv2_ref.at[src],
                dst_ref=recv2_ref.at[src],
                send_sem=p2_send_sems.at[src],
                recv_sem=p2_recv_sems.at[src],
                device_id=(src,),
                device_id_type=pl.DeviceIdType.MESH,
            ).wait_recv()
            acc_ref[src // CPB, pl.ds(lax.rem(src, CPB) * SEG, SEG), :] = (
                recv2_ref[src])
        for rdma in p2_sends:
            rdma.wait_send()

        out_dma = pltpu.make_async_copy(acc_ref, out_ref, out_sem)
        out_dma.start()
        out_dma.wait()

    hbm = pltpu.MemorySpace.HBM
    x_c = pltpu.with_memory_space_constraint(x, hbm)
    wq_c = pltpu.with_memory_space_constraint(Wq, hbm)
    wo_c = pltpu.with_memory_space_constraint(Wo, hbm)
    return pl.pallas_call(
        body,
        out_shape=jax.ShapeDtypeStruct((B, Sq, Dout), jnp.float32),
        in_specs=[
            pl.BlockSpec(memory_space=pl.ANY),
            pl.BlockSpec(memory_space=pl.ANY),
            pl.BlockSpec(memory_space=pltpu.VMEM),
            pl.BlockSpec(memory_space=pltpu.VMEM),
            pl.BlockSpec(memory_space=pl.ANY),
        ],
        out_specs=pl.BlockSpec(memory_space=pl.ANY),
        scratch_shapes=[
            pltpu.VMEM((B, Sq, Din), jnp.float32),
            pltpu.VMEM((Din, F), jnp.float32),
            pltpu.VMEM((F, Dout), jnp.float32),
            pltpu.VMEM((B, Sq, Dout), jnp.float32),
            pltpu.VMEM((N_DEV, SEG, Dout), jnp.float32),
            pltpu.VMEM((N_DEV, SEG, Dout), jnp.float32),
            pltpu.SemaphoreType.DMA((3,)),
            pltpu.SemaphoreType.DMA,
            pltpu.SemaphoreType.DMA((N_DEV,)),
            pltpu.SemaphoreType.DMA((N_DEV,)),
            pltpu.SemaphoreType.DMA((N_DEV,)),
            pltpu.SemaphoreType.DMA((N_DEV,)),
        ],
        compiler_params=pltpu.CompilerParams(collective_id=0),
    )(x_c, wq_c, K_ext.reshape(B, Skv, F), V_ext.reshape(B, Skv, F), wo_c)
